# baseline (speedup 1.0000x reference)
import dataclasses
import functools

import jax
import jax.numpy as jnp
from jax import lax
from jax.experimental import pallas as pl
from jax.experimental.pallas import tpu as pltpu
from jax.experimental.pallas import tpu_sc as plsc

_N = 10000
_E = 320000
_D_IN = 128
_HEADS = 8
_HID = 8
_D_OUT = 64
_F = _HEADS * _HID
_ROW = 128

_NC = 2
_NS = 16
_NW = _NC * _NS
_B = 128
_NBLK = _E // _B
_BLK_PER_TILE = -(-_NBLK // _NW)
_RPT = (_N // _NS) & ~7
_RTAIL = _N - _RPT * _NS


def _make_edge_pass():
    mesh = plsc.VectorSubcoreMesh(core_axis_name="c", subcore_axis_name="s")
    cp = pltpu.CompilerParams()
    if "needs_layout_passes" in pltpu.CompilerParams.__dataclass_fields__:
        cp = dataclasses.replace(cp, needs_layout_passes=False)

    @functools.partial(
        pl.kernel,
        mesh=mesh,
        compiler_params=cp,
        out_type=jax.ShapeDtypeStruct((_NC, _N, _ROW), jnp.float32),
        scratch_types=[
            pltpu.VMEM((_B,), jnp.int32),
            pltpu.VMEM((_B,), jnp.int32),
            pltpu.VMEM((_B, _ROW), jnp.float32),
            pltpu.VMEM((_B, _ROW), jnp.float32),
            pltpu.VMEM((_B, _ROW), jnp.float32),
            pltpu.VMEM_SHARED((_N, _ROW), jnp.float32),
            pltpu.SemaphoreType.DMA,
            pltpu.SemaphoreType.DMA,
        ],
    )
    def kernel(src_tab_hbm, dst_tab_hbm, src_hbm, dst_hbm, zero_hbm, out_hbm,
               src_v, dst_v, rs_v, rd_v, pay_v, acc_sh, sem_a, sem_b):
        cid = lax.axis_index("c")
        sid = lax.axis_index("s")
        wid = sid * _NC + cid

        row0 = pl.multiple_of(sid * _RPT, 8)
        pltpu.sync_copy(zero_hbm.at[pl.ds(row0, _RPT)],
                        acc_sh.at[pl.ds(row0, _RPT)])

        @pl.when(sid == 0)
        def _():
            pltpu.sync_copy(zero_hbm.at[pl.ds(_RPT * _NS, _RTAIL)],
                            acc_sh.at[pl.ds(_RPT * _NS, _RTAIL)])

        plsc.subcore_barrier()

        @pl.loop(0, _BLK_PER_TILE)
        def do_block(blk):
            blkid = blk * _NW + wid

            @pl.when(blkid < _NBLK)
            def _():
                off = pl.multiple_of(blkid * _B, 8)
                pltpu.sync_copy(src_hbm.at[pl.ds(off, _B)], src_v)
                pltpu.sync_copy(dst_hbm.at[pl.ds(off, _B)], dst_v)
                cp_a = pltpu.async_copy(src_tab_hbm.at[src_v], rs_v, sem_a)
                cp_b = pltpu.async_copy(dst_tab_hbm.at[dst_v], rd_v, sem_b)
                cp_a.wait()
                cp_b.wait()

                @plsc.parallel_loop(0, _B, unroll=4)
                def per_edge(e):
                    for j in range(4):
                        sl = pl.ds(16 * j, 16)
                        sh = pl.ds(_F + 16 * j, 16)
                        t = rs_v[e, sh] + rd_v[e, sl]
                        t = jnp.maximum(t, 0.2 * t)
                        w = jnp.exp(t)
                        pay_v[e, sh] = w
                        pay_v[e, sl] = rs_v[e, sl] * w

                pltpu.sync_copy(pay_v, acc_sh.at[dst_v], add=True)

        plsc.subcore_barrier()
        pltpu.sync_copy(acc_sh.at[pl.ds(row0, _RPT)],
                        out_hbm.at[cid].at[pl.ds(row0, _RPT)])

        @pl.when(sid == 0)
        def _():
            pltpu.sync_copy(acc_sh.at[pl.ds(_RPT * _NS, _RTAIL)],
                            out_hbm.at[cid].at[pl.ds(_RPT * _NS, _RTAIL)])

    return kernel


_edge_pass = _make_edge_pass()


def _tc_prep1_body(x_ref, w_ref, ps_ref, pd_ref, stab_ref, dtab_ref):
    h = jnp.dot(x_ref[...], w_ref[...], preferred_element_type=jnp.float32)
    stab_ref[...] = jnp.dot(h, ps_ref[...], preferred_element_type=jnp.float32)
    dtab_ref[...] = jnp.dot(h, pd_ref[...], preferred_element_type=jnp.float32)


def _tc_mid_body(p_ref, b1_ref, w2_ref, ps_ref, pd_ref, stab_ref, dtab_ref):
    acc = p_ref[0] + p_ref[1]
    out1 = acc[:, 0:_F] / (acc[:, _F:_ROW] + 1e-16) + b1_ref[...]
    x2 = jnp.where(out1 > 0, out1, jnp.exp(jnp.minimum(out1, 0.0)) - 1.0)
    h2 = jnp.dot(x2, w2_ref[...], preferred_element_type=jnp.float32)
    stab_ref[...] = jnp.dot(h2, ps_ref[...], preferred_element_type=jnp.float32)
    dtab_ref[...] = jnp.dot(h2, pd_ref[...], preferred_element_type=jnp.float32)


def _tc_final_body(p_ref, b2_ref, o_ref):
    acc = p_ref[0] + p_ref[1]
    out = acc[:, 0:_F] / (acc[:, _F:_ROW] + 1e-16) + b2_ref[...]
    m = jnp.max(out, axis=1, keepdims=True)
    z = out - m
    lse = jnp.log(jnp.sum(jnp.exp(z), axis=1, keepdims=True))
    o_ref[...] = z - lse


_BN = 1000


def kernel(x, edge_index, W1, att_src1, att_dst1, b1, W2, att_src2,
           att_dst2, b2):
    f32 = jnp.float32
    src = edge_index[0]
    dst = edge_index[1]
    zeros = jnp.zeros((_N, _ROW), f32)

    eye64 = jnp.eye(_F, dtype=f32)
    blk8 = jnp.repeat(jnp.eye(_HEADS, dtype=f32), _HID, axis=1)
    a_src = (att_src1[:, :, None] * jnp.eye(_HEADS, dtype=f32)[:, None, :]
             ).reshape(_F, _HEADS) @ blk8
    a_dst = (att_dst1[:, :, None] * jnp.eye(_HEADS, dtype=f32)[:, None, :]
             ).reshape(_F, _HEADS) @ blk8
    z64 = jnp.zeros((_F, _F), f32)
    PS1 = jnp.concatenate([eye64, a_src], axis=1)
    PD1 = jnp.concatenate([a_dst, z64], axis=1)
    PS2 = jnp.concatenate([eye64, jnp.tile(att_src2.T, (1, _F))], axis=1)
    PD2 = jnp.concatenate([jnp.tile(att_dst2.T, (1, _F)), z64], axis=1)

    grid = (_N // _BN,)
    full = lambda shape: pl.BlockSpec(shape, lambda i: tuple(0 for _ in shape))
    rows = lambda w: pl.BlockSpec((_BN, w), lambda i: (i, 0))
    part = pl.BlockSpec((_NC, _BN, _ROW), lambda i: (0, i, 0))

    stab1, dtab1 = pl.pallas_call(
        _tc_prep1_body,
        grid=grid,
        in_specs=[rows(_D_IN), full((_D_IN, _F)), full((_F, _ROW)),
                  full((_F, _ROW))],
        out_specs=[rows(_ROW), rows(_ROW)],
        out_shape=[jax.ShapeDtypeStruct((_N, _ROW), f32),
                   jax.ShapeDtypeStruct((_N, _ROW), f32)],
    )(x, W1, PS1, PD1)

    part1 = _edge_pass(stab1, dtab1, src, dst, zeros)

    stab2, dtab2 = pl.pallas_call(
        _tc_mid_body,
        grid=grid,
        in_specs=[part, full((1, _F)), full((_F, _F)), full((_F, _ROW)),
                  full((_F, _ROW))],
        out_specs=[rows(_ROW), rows(_ROW)],
        out_shape=[jax.ShapeDtypeStruct((_N, _ROW), f32),
                   jax.ShapeDtypeStruct((_N, _ROW), f32)],
    )(part1, b1.reshape(1, _F), W2, PS2, PD2)

    part2 = _edge_pass(stab2, dtab2, src, dst, zeros)

    out = pl.pallas_call(
        _tc_final_body,
        grid=grid,
        in_specs=[part, full((1, _F))],
        out_specs=rows(_F),
        out_shape=jax.ShapeDtypeStruct((_N, _D_OUT), f32),
    )(part2, b2.reshape(1, _F))

    return out


# speedup vs baseline: 1.1105x; 1.1105x over previous
import dataclasses
import functools

import jax
import jax.numpy as jnp
from jax import lax
from jax.experimental import pallas as pl
from jax.experimental.pallas import tpu as pltpu
from jax.experimental.pallas import tpu_sc as plsc

_N = 10000
_E = 320000
_D_IN = 128
_HEADS = 8
_HID = 8
_D_OUT = 64
_F = _HEADS * _HID
_ROW = 128
_AS_OFF = 64
_AD_OFF = 80
_PAY = 80

_NC = 2
_NS = 16
_NW = _NC * _NS
_B = 128
_NBLK = _E // _B
_BLK_PER_TILE = -(-_NBLK // _NW)
_RPT = (_N // _NS) & ~7
_RTAIL = _N - _RPT * _NS


def _edge_pass(layer1: bool):
    mesh = plsc.VectorSubcoreMesh(core_axis_name="c", subcore_axis_name="s")
    cp = pltpu.CompilerParams()
    if "needs_layout_passes" in pltpu.CompilerParams.__dataclass_fields__:
        cp = dataclasses.replace(cp, needs_layout_passes=False)

    @functools.partial(
        pl.kernel,
        mesh=mesh,
        compiler_params=cp,
        out_type=jax.ShapeDtypeStruct((_NC, _N, _PAY), jnp.float32),
        scratch_types=[
            pltpu.VMEM((_B,), jnp.int32),
            pltpu.VMEM((_B,), jnp.int32),
            pltpu.VMEM((_B, _ROW), jnp.float32),
            pltpu.VMEM((_B, _ROW), jnp.float32),
            pltpu.VMEM((_B, _PAY), jnp.float32),
            pltpu.VMEM_SHARED((_N, _PAY), jnp.float32),
            pltpu.SemaphoreType.DMA,
            pltpu.SemaphoreType.DMA,
        ],
    )
    def kernel(tab_hbm, src_hbm, dst_hbm, zero_hbm, out_hbm,
               src_v, dst_v, rs_v, rd_v, pay_v, acc_sh, sem_a, sem_b):
        cid = lax.axis_index("c")
        sid = lax.axis_index("s")
        wid = sid * _NC + cid

        row0 = pl.multiple_of(sid * _RPT, 8)
        pltpu.sync_copy(zero_hbm.at[pl.ds(row0, _RPT)],
                        acc_sh.at[pl.ds(row0, _RPT)])

        @pl.when(sid == 0)
        def _():
            pltpu.sync_copy(zero_hbm.at[pl.ds(_RPT * _NS, _RTAIL)],
                            acc_sh.at[pl.ds(_RPT * _NS, _RTAIL)])

        plsc.subcore_barrier()

        lanes = lax.iota(jnp.int32, 16)
        hi8 = (lanes >= 8).astype(jnp.int32)

        @pl.loop(0, _BLK_PER_TILE)
        def do_block(blk):
            blkid = blk * _NW + wid

            @pl.when(blkid < _NBLK)
            def _():
                off = pl.multiple_of(blkid * _B, 8)
                pltpu.sync_copy(src_hbm.at[pl.ds(off, _B)], src_v)
                pltpu.sync_copy(dst_hbm.at[pl.ds(off, _B)], dst_v)
                cp_a = pltpu.async_copy(tab_hbm.at[src_v], rs_v, sem_a)
                cp_b = pltpu.async_copy(tab_hbm.at[dst_v], rd_v, sem_b)
                cp_a.wait()
                cp_b.wait()

                @plsc.parallel_loop(0, _B, unroll=4)
                def per_edge(e):
                    a_sum = (rs_v[e, pl.ds(_AS_OFF, 16)]
                             + rd_v[e, pl.ds(_AD_OFF, 16)])
                    a_act = jnp.maximum(a_sum, 0.2 * a_sum)
                    ex = jnp.exp(a_act)
                    pay_v[e, pl.ds(_F, 16)] = ex
                    if layer1:
                        evec = jnp.full((16,), e, dtype=jnp.int32)
                        for j in range(4):
                            idx = hi8 + (_F + 2 * j)
                            bj = plsc.load_gather(pay_v, [evec, idx])
                            pay_v[e, pl.ds(16 * j, 16)] = (
                                rs_v[e, pl.ds(16 * j, 16)] * bj)
                    else:
                        for j in range(4):
                            pay_v[e, pl.ds(16 * j, 16)] = (
                                rs_v[e, pl.ds(16 * j, 16)] * ex)

                pltpu.sync_copy(pay_v, acc_sh.at[dst_v], add=True)

        plsc.subcore_barrier()
        pltpu.sync_copy(acc_sh.at[pl.ds(row0, _RPT)],
                        out_hbm.at[cid].at[pl.ds(row0, _RPT)])

        @pl.when(sid == 0)
        def _():
            pltpu.sync_copy(acc_sh.at[pl.ds(_RPT * _NS, _RTAIL)],
                            out_hbm.at[cid].at[pl.ds(_RPT * _NS, _RTAIL)])

    return kernel


_edge_pass_l1 = _edge_pass(True)
_edge_pass_l2 = _edge_pass(False)


def _tc_prep1_body(x_ref, w_ref, p_ref, tab_ref):
    h = jnp.dot(x_ref[...], w_ref[...], preferred_element_type=jnp.float32)
    tab_ref[...] = jnp.dot(h, p_ref[...], preferred_element_type=jnp.float32)


def _tc_mid_body(p_ref, b1_ref, w2_ref, p2_ref, b8_ref, tab_ref):
    acc = p_ref[0] + p_ref[1]
    den = jnp.dot(acc, b8_ref[...], preferred_element_type=jnp.float32)
    out1 = acc[:, 0:_F] / (den + 1e-16) + b1_ref[...]
    x2 = jnp.where(out1 > 0, out1, jnp.exp(jnp.minimum(out1, 0.0)) - 1.0)
    h2 = jnp.dot(x2, w2_ref[...], preferred_element_type=jnp.float32)
    tab_ref[...] = jnp.dot(h2, p2_ref[...], preferred_element_type=jnp.float32)


def _tc_final_body(p_ref, b2_ref, o_ref):
    acc = p_ref[0] + p_ref[1]
    out = acc[:, 0:_F] / (acc[:, _F:_F + 1] + 1e-16) + b2_ref[...]
    m = jnp.max(out, axis=1, keepdims=True)
    z = out - m
    lse = jnp.log(jnp.sum(jnp.exp(z), axis=1, keepdims=True))
    o_ref[...] = z - lse


_BN = 1000


def kernel(x, edge_index, W1, att_src1, att_dst1, b1, W2, att_src2,
           att_dst2, b2):
    f32 = jnp.float32
    src = edge_index[0]
    dst = edge_index[1]
    zeros = jnp.zeros((_N, _PAY), f32)

    eye64 = jnp.eye(_F, dtype=f32)
    a_src = (att_src1[:, :, None] * jnp.eye(_HEADS, dtype=f32)[:, None, :]
             ).reshape(_F, _HEADS)
    a_dst = (att_dst1[:, :, None] * jnp.eye(_HEADS, dtype=f32)[:, None, :]
             ).reshape(_F, _HEADS)
    pad8 = jnp.zeros((_F, 8), f32)
    pad32 = jnp.zeros((_F, 32), f32)
    P1 = jnp.concatenate([eye64, a_src, pad8, a_dst, pad8, pad32], axis=1)
    P2 = jnp.concatenate([eye64, jnp.tile(att_src2.T, (1, 16)),
                          jnp.tile(att_dst2.T, (1, 16)), pad32], axis=1)
    blk8 = jnp.repeat(jnp.eye(_HEADS, dtype=f32), _HID, axis=1)
    B8 = jnp.concatenate([jnp.zeros((_F, _F), f32), blk8,
                          jnp.zeros((8, _F), f32)], axis=0)

    grid = (_N // _BN,)
    full = lambda shape: pl.BlockSpec(shape, lambda i: tuple(0 for _ in shape))
    rows = lambda w: pl.BlockSpec((_BN, w), lambda i: (i, 0))
    part = pl.BlockSpec((_NC, _BN, _PAY), lambda i: (0, i, 0))

    tab1 = pl.pallas_call(
        _tc_prep1_body,
        grid=grid,
        in_specs=[rows(_D_IN), full((_D_IN, _F)), full((_F, _ROW))],
        out_specs=rows(_ROW),
        out_shape=jax.ShapeDtypeStruct((_N, _ROW), f32),
    )(x, W1, P1)

    part1 = _edge_pass_l1(tab1, src, dst, zeros)

    tab2 = pl.pallas_call(
        _tc_mid_body,
        grid=grid,
        in_specs=[part, full((1, _F)), full((_F, _F)), full((_F, _ROW)),
                  full((_PAY, _F))],
        out_specs=rows(_ROW),
        out_shape=jax.ShapeDtypeStruct((_N, _ROW), f32),
    )(part1, b1.reshape(1, _F), W2, P2, B8)

    part2 = _edge_pass_l2(tab2, src, dst, zeros)

    out = pl.pallas_call(
        _tc_final_body,
        grid=grid,
        in_specs=[part, full((1, _F))],
        out_specs=rows(_F),
        out_shape=jax.ShapeDtypeStruct((_N, _D_OUT), f32),
    )(part2, b2.reshape(1, _F))

    return out


# speedup vs baseline: 1.1144x; 1.0035x over previous
import dataclasses
import functools

import jax
import jax.numpy as jnp
from jax import lax
from jax.experimental import pallas as pl
from jax.experimental.pallas import tpu as pltpu
from jax.experimental.pallas import tpu_sc as plsc

_N = 10000
_E = 320000
_D_IN = 128
_HEADS = 8
_HID = 8
_D_OUT = 64
_F = _HEADS * _HID
_ROW = 128
_AS_OFF = 64
_AD_OFF = 80
_PAY = 80

_NC = 2
_NS = 16
_NW = _NC * _NS
_B = 128
_NBLK = _E // _B
_BLK_PER_TILE = -(-_NBLK // _NW)
_RPT = (_N // _NS) & ~7
_RTAIL = _N - _RPT * _NS


def _edge_pass(layer1: bool):
    mesh = plsc.VectorSubcoreMesh(core_axis_name="c", subcore_axis_name="s")
    cp = pltpu.CompilerParams()
    if "needs_layout_passes" in pltpu.CompilerParams.__dataclass_fields__:
        cp = dataclasses.replace(cp, needs_layout_passes=False)

    @functools.partial(
        pl.kernel,
        mesh=mesh,
        compiler_params=cp,
        out_type=jax.ShapeDtypeStruct((_NC, _N, _PAY), jnp.float32),
        scratch_types=[
            pltpu.VMEM((_B,), jnp.int32),
            pltpu.VMEM((_B,), jnp.int32),
            pltpu.VMEM((_B, _ROW), jnp.float32),
            pltpu.VMEM((_B, _ROW), jnp.float32),
            pltpu.VMEM((_B, _PAY), jnp.float32),
            pltpu.VMEM_SHARED((_N, _PAY), jnp.float32),
            pltpu.SemaphoreType.DMA,
            pltpu.SemaphoreType.DMA,
        ],
    )
    def kernel(tab_hbm, src_hbm, dst_hbm, zero_hbm, out_hbm,
               src_v, dst_v, rs_v, rd_v, pay_v, acc_sh, sem_a, sem_b):
        cid = lax.axis_index("c")
        sid = lax.axis_index("s")
        wid = sid * _NC + cid

        row0 = pl.multiple_of(sid * _RPT, 8)
        pltpu.sync_copy(zero_hbm.at[pl.ds(row0, _RPT)],
                        acc_sh.at[pl.ds(row0, _RPT)])

        @pl.when(sid == 0)
        def _():
            pltpu.sync_copy(zero_hbm.at[pl.ds(_RPT * _NS, _RTAIL)],
                            acc_sh.at[pl.ds(_RPT * _NS, _RTAIL)])

        plsc.subcore_barrier()

        lanes = lax.iota(jnp.int32, 16)
        hi8 = (lanes >= 8).astype(jnp.int32)

        @pl.loop(0, _BLK_PER_TILE)
        def do_block(blk):
            blkid = blk * _NW + wid

            @pl.when(blkid < _NBLK)
            def _():
                off = pl.multiple_of(blkid * _B, 8)
                pltpu.sync_copy(src_hbm.at[pl.ds(off, _B)], src_v)
                pltpu.sync_copy(dst_hbm.at[pl.ds(off, _B)], dst_v)
                cp_a = pltpu.async_copy(tab_hbm.at[src_v], rs_v, sem_a)
                cp_b = pltpu.async_copy(tab_hbm.at[dst_v], rd_v, sem_b)
                cp_a.wait()
                cp_b.wait()

                @plsc.parallel_loop(0, _B, unroll=8)
                def per_edge(e):
                    a_sum = (rs_v[e, pl.ds(_AS_OFF, 16)]
                             + rd_v[e, pl.ds(_AD_OFF, 16)])
                    a_act = jnp.maximum(a_sum, 0.2 * a_sum)
                    ex = jnp.exp(a_act)
                    pay_v[e, pl.ds(_F, 16)] = ex
                    if layer1:
                        evec = jnp.full((16,), e, dtype=jnp.int32)
                        for j in range(4):
                            idx = hi8 + (_F + 2 * j)
                            bj = plsc.load_gather(pay_v, [evec, idx])
                            pay_v[e, pl.ds(16 * j, 16)] = (
                                rs_v[e, pl.ds(16 * j, 16)] * bj)
                    else:
                        for j in range(4):
                            pay_v[e, pl.ds(16 * j, 16)] = (
                                rs_v[e, pl.ds(16 * j, 16)] * ex)

                pltpu.sync_copy(pay_v, acc_sh.at[dst_v], add=True)

        plsc.subcore_barrier()
        pltpu.sync_copy(acc_sh.at[pl.ds(row0, _RPT)],
                        out_hbm.at[cid].at[pl.ds(row0, _RPT)])

        @pl.when(sid == 0)
        def _():
            pltpu.sync_copy(acc_sh.at[pl.ds(_RPT * _NS, _RTAIL)],
                            out_hbm.at[cid].at[pl.ds(_RPT * _NS, _RTAIL)])

    return kernel


_edge_pass_l1 = _edge_pass(True)
_edge_pass_l2 = _edge_pass(False)


def _tc_prep1_body(x_ref, w_ref, p_ref, tab_ref):
    h = jnp.dot(x_ref[...], w_ref[...], preferred_element_type=jnp.float32)
    tab_ref[...] = jnp.dot(h, p_ref[...], preferred_element_type=jnp.float32)


def _tc_mid_body(p_ref, b1_ref, w2_ref, p2_ref, b8_ref, tab_ref):
    acc = p_ref[0] + p_ref[1]
    den = jnp.dot(acc, b8_ref[...], preferred_element_type=jnp.float32)
    out1 = acc[:, 0:_F] / (den + 1e-16) + b1_ref[...]
    x2 = jnp.where(out1 > 0, out1, jnp.exp(jnp.minimum(out1, 0.0)) - 1.0)
    h2 = jnp.dot(x2, w2_ref[...], preferred_element_type=jnp.float32)
    tab_ref[...] = jnp.dot(h2, p2_ref[...], preferred_element_type=jnp.float32)


def _tc_final_body(p_ref, b2_ref, o_ref):
    acc = p_ref[0] + p_ref[1]
    out = acc[:, 0:_F] / (acc[:, _F:_F + 1] + 1e-16) + b2_ref[...]
    m = jnp.max(out, axis=1, keepdims=True)
    z = out - m
    lse = jnp.log(jnp.sum(jnp.exp(z), axis=1, keepdims=True))
    o_ref[...] = z - lse


_BN = 1000


def kernel(x, edge_index, W1, att_src1, att_dst1, b1, W2, att_src2,
           att_dst2, b2):
    f32 = jnp.float32
    src = edge_index[0]
    dst = edge_index[1]
    zeros = jnp.zeros((_N, _PAY), f32)

    eye64 = jnp.eye(_F, dtype=f32)
    a_src = (att_src1[:, :, None] * jnp.eye(_HEADS, dtype=f32)[:, None, :]
             ).reshape(_F, _HEADS)
    a_dst = (att_dst1[:, :, None] * jnp.eye(_HEADS, dtype=f32)[:, None, :]
             ).reshape(_F, _HEADS)
    pad8 = jnp.zeros((_F, 8), f32)
    pad32 = jnp.zeros((_F, 32), f32)
    P1 = jnp.concatenate([eye64, a_src, pad8, a_dst, pad8, pad32], axis=1)
    P2 = jnp.concatenate([eye64, jnp.tile(att_src2.T, (1, 16)),
                          jnp.tile(att_dst2.T, (1, 16)), pad32], axis=1)
    blk8 = jnp.repeat(jnp.eye(_HEADS, dtype=f32), _HID, axis=1)
    B8 = jnp.concatenate([jnp.zeros((_F, _F), f32), blk8,
                          jnp.zeros((8, _F), f32)], axis=0)

    grid = (_N // _BN,)
    full = lambda shape: pl.BlockSpec(shape, lambda i: tuple(0 for _ in shape))
    rows = lambda w: pl.BlockSpec((_BN, w), lambda i: (i, 0))
    part = pl.BlockSpec((_NC, _BN, _PAY), lambda i: (0, i, 0))

    tab1 = pl.pallas_call(
        _tc_prep1_body,
        grid=grid,
        in_specs=[rows(_D_IN), full((_D_IN, _F)), full((_F, _ROW))],
        out_specs=rows(_ROW),
        out_shape=jax.ShapeDtypeStruct((_N, _ROW), f32),
    )(x, W1, P1)

    part1 = _edge_pass_l1(tab1, src, dst, zeros)

    tab2 = pl.pallas_call(
        _tc_mid_body,
        grid=grid,
        in_specs=[part, full((1, _F)), full((_F, _F)), full((_F, _ROW)),
                  full((_PAY, _F))],
        out_specs=rows(_ROW),
        out_shape=jax.ShapeDtypeStruct((_N, _ROW), f32),
    )(part1, b1.reshape(1, _F), W2, P2, B8)

    part2 = _edge_pass_l2(tab2, src, dst, zeros)

    out = pl.pallas_call(
        _tc_final_body,
        grid=grid,
        in_specs=[part, full((1, _F))],
        out_specs=rows(_F),
        out_shape=jax.ShapeDtypeStruct((_N, _D_OUT), f32),
    )(part2, b2.reshape(1, _F))

    return out
